# baseline (speedup 1.0000x reference)
import functools

import jax
import jax.numpy as jnp
from jax import lax
from jax.experimental import pallas as pl
from jax.experimental.pallas import tpu as pltpu
from jax.experimental.pallas import tpu_sc as plsc

N_ROWS = 1_000_000
N_COLS = 32
N_IDX = 16_384

_NDMA = 8
_RPD = N_ROWS // _NDMA


def _copy_body(x_hbm, o_hbm, sem):
  copies = [
      pltpu.async_copy(
          x_hbm.at[pl.ds(i * _RPD, _RPD)], o_hbm.at[pl.ds(i * _RPD, _RPD)], sem
      )
      for i in range(_NDMA)
  ]
  for c in copies:
    c.wait()


def _tc_copy(x):
  return pl.pallas_call(
      _copy_body,
      in_specs=[pl.BlockSpec(memory_space=pl.ANY)],
      out_specs=pl.BlockSpec(memory_space=pl.ANY),
      out_shape=jax.ShapeDtypeStruct((N_ROWS, N_COLS), jnp.float32),
      scratch_shapes=[pltpu.SemaphoreType.DMA],
  )(x)


_NW = 32
_CPW = N_IDX // _NW
_CHUNK = 128
_NCH = _CPW // _CHUNK

_sc_mesh = plsc.VectorSubcoreMesh(core_axis_name="c", subcore_axis_name="s")


@functools.partial(
    pl.kernel,
    out_type=(),
    mesh=_sc_mesh,
    compiler_params=pltpu.CompilerParams(use_tc_tiling_on_sc=False),
    scratch_types=[
        pltpu.VMEM((_NCH, _CHUNK), jnp.int32),
        pltpu.VMEM((_CPW, N_COLS), jnp.float32),
        pltpu.SemaphoreType.DMA,
    ],
)
def _sc_scatter(out_ref, idx2_hbm, y_hbm, idx_v, rows_v, sem):
  wid = lax.axis_index("c") * 16 + lax.axis_index("s")
  base = wid * _CPW
  pltpu.sync_copy(idx2_hbm.at[pl.ds(wid * _NCH, _NCH)], idx_v)
  pltpu.sync_copy(y_hbm.at[pl.ds(base, _CPW)], rows_v)
  copies = []
  for j in range(_NCH):
    copies.append(
        pltpu.async_copy(
            rows_v.at[pl.ds(j * _CHUNK, _CHUNK)], out_ref.at[idx_v.at[j]], sem
        )
    )
  for c in copies:
    c.wait()


def kernel(dim, x, index, y):
  idx = index + jnp.asarray(dim, index.dtype)
  idx2 = idx.reshape(N_IDX // _CHUNK, _CHUNK)
  out0 = _tc_copy(x)
  ref = jax.new_ref(out0)
  _sc_scatter(ref, idx2, y)
  return jax.freeze(ref)


# speedup vs baseline: 15.2141x; 15.2141x over previous
import functools

import jax
import jax.numpy as jnp
from jax import lax
from jax.experimental import pallas as pl
from jax.experimental.pallas import tpu as pltpu
from jax.experimental.pallas import tpu_sc as plsc

N_ROWS = 1_000_000
N_COLS = 32
N_IDX = 16_384

_WIDE_ROWS = N_ROWS * N_COLS // 128
_BR = 10_000


def _copy_body(x_ref, o_ref):
  o_ref[...] = x_ref[...]


def _tc_copy(x):
  x2 = x.reshape(_WIDE_ROWS, 128)
  out2 = pl.pallas_call(
      _copy_body,
      grid=(_WIDE_ROWS // _BR,),
      in_specs=[pl.BlockSpec((_BR, 128), lambda i: (i, 0))],
      out_specs=pl.BlockSpec((_BR, 128), lambda i: (i, 0)),
      out_shape=jax.ShapeDtypeStruct((_WIDE_ROWS, 128), jnp.float32),
  )(x2)
  return out2.reshape(N_ROWS, N_COLS)


_NW = 32
_CPW = N_IDX // _NW
_CHUNK = 128
_NCH = _CPW // _CHUNK

_sc_mesh = plsc.VectorSubcoreMesh(core_axis_name="c", subcore_axis_name="s")


@functools.partial(
    pl.kernel,
    out_type=(),
    mesh=_sc_mesh,
    compiler_params=pltpu.CompilerParams(use_tc_tiling_on_sc=False),
    scratch_types=[
        pltpu.VMEM((_NCH, _CHUNK), jnp.int32),
        pltpu.VMEM((_CPW, N_COLS), jnp.float32),
        pltpu.SemaphoreType.DMA,
    ],
)
def _sc_scatter(out_ref, idx2_hbm, y_hbm, idx_v, rows_v, sem):
  wid = lax.axis_index("c") * 16 + lax.axis_index("s")
  base = wid * _CPW
  pltpu.sync_copy(idx2_hbm.at[pl.ds(wid * _NCH, _NCH)], idx_v)
  pltpu.sync_copy(y_hbm.at[pl.ds(base, _CPW)], rows_v)
  copies = []
  for j in range(_NCH):
    copies.append(
        pltpu.async_copy(
            rows_v.at[pl.ds(j * _CHUNK, _CHUNK)], out_ref.at[idx_v.at[j]], sem
        )
    )
  for c in copies:
    c.wait()


def kernel(dim, x, index, y):
  idx = index + jnp.asarray(dim, index.dtype)
  idx2 = idx.reshape(N_IDX // _CHUNK, _CHUNK)
  out0 = _tc_copy(x)
  ref = jax.new_ref(out0)
  _sc_scatter(ref, idx2, y)
  return jax.freeze(ref)
